# Initial kernel scaffold; baseline (speedup 1.0000x reference)
import jax
import jax.numpy as jnp
from jax.experimental import pallas as pl


def kernel(phi, succ_feats, W):
    raise NotImplementedError("write your pallas kernel here")


# speedup vs baseline: 4.4055x; 4.4055x over previous
import jax
import jax.numpy as jnp
from jax import lax
from jax.experimental import pallas as pl
from jax.experimental.pallas import tpu as pltpu
from jax.experimental.pallas import tpu_sc as plsc

N_FEATURES = 6
T = 0.001
B = 4096
L = 16
NC, NS = 2, 16
NW = NC * NS
BPW = B // NW
GROUPS = BPW // L
AUX_ROWS = 2 * N_FEATURES + N_FEATURES


def _sc_body(phi_hbm, aux_hbm, out_hbm, phi_v, aux_v, tab_v, out_v):
    wid = lax.axis_index("s") * NC + lax.axis_index("c")
    base = wid * BPW

    pltpu.sync_copy(phi_hbm.at[:, pl.ds(base, BPW)], phi_v)
    pltpu.sync_copy(aux_hbm, aux_v)

    lanes = pl.ds(0, L)
    wvec = [aux_v[2 * N_FEATURES + c, lanes] for c in range(N_FEATURES)]

    v0 = jnp.zeros((L,), jnp.float32)
    v1 = jnp.zeros((L,), jnp.float32)
    for c in range(N_FEATURES):
        v0 = v0 + aux_v[c, lanes] * wvec[c]
        v1 = v1 + aux_v[N_FEATURES + c, lanes] * wvec[c]
    m = jnp.maximum(v0, v1)
    e0 = jnp.exp((v0 - m) / T)
    e1 = jnp.exp((v1 - m) / T)
    tab_v[...] = (e0 * v0 + e1 * v1) / (e0 + e1)

    for g in range(GROUPS):
        sl = pl.ds(g * L, L)
        delta = []
        for p in range(2):
            row = p * 10
            pr = jnp.zeros((L,), jnp.float32)
            for c in range(N_FEATURES):
                pr = pr + phi_v[row + c, sl].astype(jnp.float32) * wvec[c]
            ix_ss = phi_v[row + 6, sl] * 4 + phi_v[row + 7, sl]
            ix_es = phi_v[row + 8, sl] * 4 + phi_v[row + 9, sl]
            v_ss = plsc.load_gather(tab_v, [ix_ss])
            v_es = plsc.load_gather(tab_v, [ix_es])
            delta.append(pr + v_es - v_ss)
        d = delta[0] - delta[1]
        out_v[0, sl] = 1.0 / (1.0 + jnp.exp(-d))
        out_v[1, sl] = 1.0 / (1.0 + jnp.exp(d))

    pltpu.sync_copy(out_v, out_hbm.at[:, pl.ds(base, BPW)])


@jax.jit
def kernel(phi, succ_feats, W):
    phi_t = jnp.transpose(phi.astype(jnp.int32), (1, 2, 0)).reshape(20, B)
    sf_part = jnp.transpose(
        succ_feats.reshape(2, L, N_FEATURES), (0, 2, 1)
    ).reshape(2 * N_FEATURES, L)
    w_part = jnp.broadcast_to(W.reshape(N_FEATURES, 1), (N_FEATURES, L))
    aux = jnp.pad(jnp.concatenate([sf_part, w_part], axis=0),
                  ((0, 0), (0, 128 - L)))

    mesh = plsc.VectorSubcoreMesh(core_axis_name="c", subcore_axis_name="s")
    run = pl.kernel(
        _sc_body,
        out_type=jax.ShapeDtypeStruct((2, B), jnp.float32),
        mesh=mesh,
        scratch_types=[
            pltpu.VMEM((20, BPW), jnp.int32),
            pltpu.VMEM((AUX_ROWS, 128), jnp.float32),
            pltpu.VMEM((L,), jnp.float32),
            pltpu.VMEM((2, BPW), jnp.float32),
        ],
        compiler_params=pltpu.CompilerParams(needs_layout_passes=False),
    )
    out2 = run(phi_t, aux)
    return jnp.transpose(out2)[:, :, None]
